# Initial kernel scaffold; baseline (speedup 1.0000x reference)
import jax
import jax.numpy as jnp
from jax.experimental import pallas as pl


def kernel(x, edge_index, edge_attr, batch, t, p, lin0_w, lin0_b, nn1_w, nn1_b, nn2_w, nn2_b, root_w, conv_b, gru_wih, gru_whh, gru_bih, gru_bhh, lstm_wih, lstm_whh, lstm_bih, lstm_bhh, lin1_w, lin1_b, lin2_w, lin2_b, lin3_w, lin3_b):
    raise NotImplementedError("write your pallas kernel here")


# speedup vs baseline: 4.6639x; 4.6639x over previous
import functools

import jax
import jax.numpy as jnp
from jax import lax
from jax.experimental import pallas as pl
from jax.experimental.pallas import tpu as pltpu
from jax.experimental.pallas import tpu_sc as plsc

N = 10000
E = 160000
DIN = 128
H = 16
B = 64
DE = 6

NC = 2
NS = 16
NW = NC * NS
CL = 128
CH = 40
PW = CH * CL
EP = NW * PW
NP = 10016
RPS = NP // NS

EPK = EP // 8
NPK = NP // 8
EB = 2048
EBK = EB // 8


def _mesh():
    return plsc.VectorSubcoreMesh(core_axis_name="c", subcore_axis_name="s")


def _sc_gather(table, idx3):
    @functools.partial(
        pl.kernel,
        mesh=_mesh(),
        out_type=jax.ShapeDtypeStruct((EP, H), jnp.float32),
        scratch_types=[
            pltpu.VMEM((CH, CL), jnp.int32),
            pltpu.VMEM((PW, H), jnp.float32),
            pltpu.SemaphoreType.DMA,
        ],
        compiler_params=pltpu.CompilerParams(use_tc_tiling_on_sc=False),
    )
    def k(table_hbm, idx_hbm, g_hbm, idx_v, rows_v, sem):
        wid = lax.axis_index("s") * NC + lax.axis_index("c")
        pltpu.sync_copy(idx_hbm.at[wid], idx_v)

        def body(j, carry):
            pltpu.async_copy(
                table_hbm.at[idx_v.at[j]], rows_v.at[pl.ds(j * CL, CL)], sem
            )
            return carry

        lax.fori_loop(0, CH, body, 0)
        pltpu.make_async_copy(g_hbm.at[pl.ds(0, PW)], rows_v, sem).wait()
        pltpu.sync_copy(rows_v, g_hbm.at[pl.ds(wid * PW, PW)])

    return k(table, idx3)


def _sc_scatter(msg, idx3, zeros_np):
    @functools.partial(
        pl.kernel,
        mesh=_mesh(),
        out_type=jax.ShapeDtypeStruct((NC, NP, H), jnp.float32),
        scratch_types=[
            pltpu.VMEM((CH, CL), jnp.int32),
            pltpu.VMEM((PW, H), jnp.float32),
            pltpu.VMEM_SHARED((NP, H), jnp.float32),
            pltpu.SemaphoreType.DMA,
        ],
        compiler_params=pltpu.CompilerParams(use_tc_tiling_on_sc=False),
    )
    def k(msg_hbm, idx_hbm, z_hbm, parts_hbm, idx_v, msg_v, agg_sh, sem):
        cid = lax.axis_index("c")
        sid = lax.axis_index("s")
        wid = sid * NC + cid
        pltpu.sync_copy(
            z_hbm.at[pl.ds(sid * RPS, RPS)], agg_sh.at[pl.ds(sid * RPS, RPS)]
        )
        pltpu.sync_copy(idx_hbm.at[wid], idx_v)
        pltpu.sync_copy(msg_hbm.at[pl.ds(wid * PW, PW)], msg_v)
        plsc.subcore_barrier()

        def body(j, carry):
            pltpu.async_copy(
                msg_v.at[pl.ds(j * CL, CL)], agg_sh.at[idx_v.at[j]], sem,
                add=True,
            )
            return carry

        lax.fori_loop(0, CH, body, 0)
        pltpu.make_async_copy(msg_hbm.at[pl.ds(0, PW)], msg_v, sem).wait()
        plsc.subcore_barrier()
        pltpu.sync_copy(
            agg_sh.at[pl.ds(sid * RPS, RPS)],
            parts_hbm.at[cid, pl.ds(sid * RPS, RPS)],
        )

    return k(msg, idx3, zeros_np)


def _tc_lin0(xp, wbd, bt):
    def body(x_ref, w_ref, b_ref, o_ref):
        o_ref[...] = jnp.maximum(
            jnp.dot(x_ref[...], w_ref[...], preferred_element_type=jnp.float32)
            + b_ref[...],
            0.0,
        )

    return pl.pallas_call(
        body,
        out_shape=jax.ShapeDtypeStruct((NPK, 128), jnp.float32),
    )(xp, wbd, bt)


def _tc_msg(ea8, gp, w1, b1, w2, b2, R, S):
    def body(ea_ref, g_ref, w1_ref, b1_ref, w2_ref, b2_ref, R_ref, S_ref, o_ref):
        ea = ea_ref[...]
        g = g_ref[...]
        pieces = []
        for j in range(8):
            h1 = jnp.maximum(
                jnp.dot(ea[:, 8 * j:8 * j + 8], w1_ref[...],
                        preferred_element_type=jnp.float32) + b1_ref[...],
                0.0,
            )
            w = (
                jnp.dot(h1, w2_ref[...], preferred_element_type=jnp.float32)
                + b2_ref[...]
            )
            gexp = jnp.dot(g[:, 16 * j:16 * j + 16], R_ref[...],
                           preferred_element_type=jnp.float32)
            pieces.append(
                jnp.dot(gexp * w, S_ref[...], preferred_element_type=jnp.float32)
            )
        o_ref[...] = jnp.concatenate(pieces, axis=1)

    return pl.pallas_call(
        body,
        grid=(EP // EB,),
        in_specs=[
            pl.BlockSpec((EBK, 64), lambda i: (i, 0)),
            pl.BlockSpec((EBK, 128), lambda i: (i, 0)),
            pl.BlockSpec((8, 64), lambda i: (0, 0)),
            pl.BlockSpec((1, 64), lambda i: (0, 0)),
            pl.BlockSpec((64, H * H), lambda i: (0, 0)),
            pl.BlockSpec((1, H * H), lambda i: (0, 0)),
            pl.BlockSpec((H, H * H), lambda i: (0, 0)),
            pl.BlockSpec((H * H, H), lambda i: (0, 0)),
        ],
        out_specs=pl.BlockSpec((EBK, 128), lambda i: (i, 0)),
        out_shape=jax.ShapeDtypeStruct((EPK, 128), jnp.float32),
    )(ea8, gp, w1, b1, w2, b2, R, S)


def _tc_gru(parts, hp, bd_root, cbt, bd_ir, bd_iz, bd_in, bd_hr, bd_hz, bd_hn,
            bih_t, bhh_t):
    def body(p_ref, h_ref, rt_ref, cb_ref, ir_ref, iz_ref, in_ref,
             hr_ref, hz_ref, hn_ref, bi_ref, bh_ref, o_ref):
        h = h_ref[...]
        agg = p_ref[0] + p_ref[1]
        m = jnp.maximum(
            agg
            + jnp.dot(h, rt_ref[...], preferred_element_type=jnp.float32)
            + cb_ref[...],
            0.0,
        )
        bi = bi_ref[...]
        bh = bh_ref[...]
        ir = jnp.dot(m, ir_ref[...], preferred_element_type=jnp.float32) + bi[:, :128]
        iz = jnp.dot(m, iz_ref[...], preferred_element_type=jnp.float32) + bi[:, 128:256]
        inn = jnp.dot(m, in_ref[...], preferred_element_type=jnp.float32) + bi[:, 256:]
        hr = jnp.dot(h, hr_ref[...], preferred_element_type=jnp.float32) + bh[:, :128]
        hz = jnp.dot(h, hz_ref[...], preferred_element_type=jnp.float32) + bh[:, 128:256]
        hn = jnp.dot(h, hn_ref[...], preferred_element_type=jnp.float32) + bh[:, 256:]
        r = jax.nn.sigmoid(ir + hr)
        z = jax.nn.sigmoid(iz + hz)
        n = jnp.tanh(inn + r * hn)
        o_ref[...] = (1.0 - z) * n + z * h

    return pl.pallas_call(
        body,
        out_shape=jax.ShapeDtypeStruct((NPK, 128), jnp.float32),
    )(parts, hp, bd_root, cbt, bd_ir, bd_iz, bd_in, bd_hr, bd_hz, bd_hn,
      bih_t, bhh_t)


def _tc_set2set(out, batch2, t, p, lstm_wih, lstm_whh, lstm_bih, lstm_bhh,
                lin1_w, lin1_b, lin2_w, lin2_b, lin3_w, lin3_b):
    def body(o_ref, b_ref, t_ref, p_ref, wih_ref, whh_ref, bih_ref, bhh_ref,
             w1_ref, b1_ref, w2_ref, b2_ref, w3_ref, b3_ref, res_ref):
        xn = o_ref[...][:N, :]
        mask = b_ref[...] == lax.broadcasted_iota(jnp.int32, (1, B), 1)
        q_star = jnp.zeros((B, 2 * H), jnp.float32)
        hs = jnp.zeros((B, H), jnp.float32)
        cs = jnp.zeros((B, H), jnp.float32)
        for _ in range(3):
            g = (
                jnp.dot(q_star, wih_ref[...], preferred_element_type=jnp.float32)
                + bih_ref[...]
                + jnp.dot(hs, whh_ref[...], preferred_element_type=jnp.float32)
                + bhh_ref[...]
            )
            ig = jax.nn.sigmoid(g[:, :H])
            fg = jax.nn.sigmoid(g[:, H:2 * H])
            gg = jnp.tanh(g[:, 2 * H:3 * H])
            og = jax.nn.sigmoid(g[:, 3 * H:])
            cs = fg * cs + ig * gg
            hs = og * jnp.tanh(cs)
            q = hs
            qb = jnp.dot(
                mask.astype(jnp.float32), q, preferred_element_type=jnp.float32
            )
            e = jnp.sum(xn * qb, axis=1, keepdims=True)
            em = jnp.where(mask, e, -jnp.inf)
            emax = jnp.max(em, axis=0, keepdims=True)
            ee = jnp.where(mask, jnp.exp(e - emax), 0.0)
            den = jnp.sum(ee, axis=0, keepdims=True)
            den = jnp.where(den == 0.0, 1.0, den)
            amat = ee / den
            rr = lax.dot_general(
                amat, xn, (((0,), (0,)), ((), ())),
                preferred_element_type=jnp.float32,
            )
            q_star = jnp.concatenate([q, rr], axis=1)
        o = jnp.concatenate([q_star, t_ref[...], p_ref[...]], axis=1)
        o = jnp.maximum(
            jnp.dot(o, w1_ref[...], preferred_element_type=jnp.float32)
            + b1_ref[...],
            0.0,
        )
        o = jnp.maximum(
            jnp.dot(o, w2_ref[...], preferred_element_type=jnp.float32)
            + b2_ref[...],
            0.0,
        )
        res_ref[...] = (
            jnp.dot(o, w3_ref[...], preferred_element_type=jnp.float32)
            + b3_ref[...]
        )

    return pl.pallas_call(
        body,
        out_shape=jax.ShapeDtypeStruct((B, 1), jnp.float32),
    )(out, batch2, t, p, lstm_wih, lstm_whh, lstm_bih, lstm_bhh,
      lin1_w, lin1_b, lin2_w, lin2_b, lin3_w, lin3_b)


def kernel(x, edge_index, edge_attr, batch, t, p,
           lin0_w, lin0_b, nn1_w, nn1_b, nn2_w, nn2_b, root_w, conv_b,
           gru_wih, gru_whh, gru_bih, gru_bhh,
           lstm_wih, lstm_whh, lstm_bih, lstm_bhh,
           lin1_w, lin1_b, lin2_w, lin2_b, lin3_w, lin3_b):
    f32 = jnp.float32
    src = edge_index[0].astype(jnp.int32)
    dst = edge_index[1].astype(jnp.int32)
    pad = EP - E
    src3 = jnp.concatenate([src, jnp.zeros((pad,), jnp.int32)]).reshape(NW, CH, CL)
    dst3 = jnp.concatenate([dst, jnp.full((pad,), N, jnp.int32)]).reshape(NW, CH, CL)
    ea8 = jnp.concatenate([edge_attr, jnp.zeros((E, 2), f32)], axis=1)
    ea8 = jnp.concatenate([ea8, jnp.zeros((pad, 8), f32)]).reshape(EPK, 64)
    nn1_w8 = jnp.concatenate([nn1_w, jnp.zeros((2, 64), f32)], axis=0)
    x_pad = jnp.concatenate([x, jnp.zeros((NP - N, DIN), f32)])
    zeros_np = jnp.zeros((NP, H), f32)

    R = jnp.kron(jnp.eye(H, dtype=f32), jnp.ones((1, H), f32))
    S = jnp.kron(jnp.ones((H, 1), f32), jnp.eye(H, dtype=f32))
    eye8 = jnp.eye(8, dtype=f32)

    outp = _tc_lin0(
        x_pad.reshape(NPK, 8 * DIN),
        jnp.kron(eye8, lin0_w),
        jnp.tile(lin0_b, 8).reshape(1, 128),
    )

    bd = lambda w: jnp.kron(eye8, w)
    gru_args = (
        bd(root_w), jnp.tile(conv_b, 8).reshape(1, 128),
        bd(gru_wih[:, :H]), bd(gru_wih[:, H:2 * H]), bd(gru_wih[:, 2 * H:]),
        bd(gru_whh[:, :H]), bd(gru_whh[:, H:2 * H]), bd(gru_whh[:, 2 * H:]),
        jnp.concatenate(
            [jnp.tile(gru_bih[g * H:(g + 1) * H], 8) for g in range(3)]
        ).reshape(1, 384),
        jnp.concatenate(
            [jnp.tile(gru_bhh[g * H:(g + 1) * H], 8) for g in range(3)]
        ).reshape(1, 384),
    )

    nn1_b2 = nn1_b.reshape(1, 64)
    nn2_b2 = nn2_b.reshape(1, H * H)
    for _ in range(4):
        g = _sc_gather(outp.reshape(NP, H), src3)
        msgp = _tc_msg(ea8, g.reshape(EPK, 128), nn1_w8, nn1_b2, nn2_w,
                       nn2_b2, R, S)
        parts = _sc_scatter(msgp.reshape(EP, H), dst3, zeros_np)
        outp = _tc_gru(parts.reshape(NC, NPK, 128), outp, *gru_args)

    res = _tc_set2set(
        outp.reshape(NP, H), batch.astype(jnp.int32).reshape(N, 1), t, p,
        lstm_wih, lstm_whh, lstm_bih.reshape(1, 4 * H), lstm_bhh.reshape(1, 4 * H),
        lin1_w, lin1_b.reshape(1, H), lin2_w, lin2_b.reshape(1, 64),
        lin3_w, lin3_b.reshape(1, 1))
    return res.reshape(-1)


# speedup vs baseline: 4.9013x; 1.0509x over previous
import functools

import jax
import jax.numpy as jnp
from jax import lax
from jax.experimental import pallas as pl
from jax.experimental.pallas import tpu as pltpu
from jax.experimental.pallas import tpu_sc as plsc

N = 10000
E = 160000
DIN = 128
H = 16
B = 64
DE = 6

NC = 2
NS = 16
NW = NC * NS
CL = 128
CH = 40
PW = CH * CL
EP = NW * PW
NP = 10016
RPS = NP // NS

EPK = EP // 8
NPK = NP // 8
EB = 2048
EBK = EB // 8


def _mesh():
    return plsc.VectorSubcoreMesh(core_axis_name="c", subcore_axis_name="s")


def _sc_gather(table, idx3):
    @functools.partial(
        pl.kernel,
        mesh=_mesh(),
        out_type=jax.ShapeDtypeStruct((EP, H), jnp.float32),
        scratch_types=[
            pltpu.VMEM((CH, CL), jnp.int32),
            pltpu.VMEM((PW, H), jnp.float32),
            pltpu.SemaphoreType.DMA,
        ],
        compiler_params=pltpu.CompilerParams(use_tc_tiling_on_sc=False),
    )
    def k(table_hbm, idx_hbm, g_hbm, idx_v, rows_v, sem):
        wid = lax.axis_index("s") * NC + lax.axis_index("c")
        pltpu.sync_copy(idx_hbm.at[wid], idx_v)

        def body(j, carry):
            pltpu.async_copy(
                table_hbm.at[idx_v.at[j]], rows_v.at[pl.ds(j * CL, CL)], sem
            )
            return carry

        lax.fori_loop(0, CH, body, 0)
        pltpu.make_async_copy(g_hbm.at[pl.ds(0, PW)], rows_v, sem).wait()
        pltpu.sync_copy(rows_v, g_hbm.at[pl.ds(wid * PW, PW)])

    return k(table, idx3)


def _sc_scatter(msg, idx3, zeros_np):
    @functools.partial(
        pl.kernel,
        mesh=_mesh(),
        out_type=jax.ShapeDtypeStruct((NC, NP, H), jnp.float32),
        scratch_types=[
            pltpu.VMEM((CH, CL), jnp.int32),
            pltpu.VMEM((PW, H), jnp.float32),
            pltpu.VMEM_SHARED((NP, H), jnp.float32),
            pltpu.SemaphoreType.DMA,
        ],
        compiler_params=pltpu.CompilerParams(use_tc_tiling_on_sc=False),
    )
    def k(msg_hbm, idx_hbm, z_hbm, parts_hbm, idx_v, msg_v, agg_sh, sem):
        cid = lax.axis_index("c")
        sid = lax.axis_index("s")
        wid = sid * NC + cid
        pltpu.sync_copy(
            z_hbm.at[pl.ds(sid * RPS, RPS)], agg_sh.at[pl.ds(sid * RPS, RPS)]
        )
        pltpu.sync_copy(idx_hbm.at[wid], idx_v)
        pltpu.sync_copy(msg_hbm.at[pl.ds(wid * PW, PW)], msg_v)
        plsc.subcore_barrier()

        def body(j, carry):
            pltpu.async_copy(
                msg_v.at[pl.ds(j * CL, CL)], agg_sh.at[idx_v.at[j]], sem,
                add=True,
            )
            return carry

        lax.fori_loop(0, CH, body, 0)
        pltpu.make_async_copy(msg_hbm.at[pl.ds(0, PW)], msg_v, sem).wait()
        plsc.subcore_barrier()
        pltpu.sync_copy(
            agg_sh.at[pl.ds(sid * RPS, RPS)],
            parts_hbm.at[cid, pl.ds(sid * RPS, RPS)],
        )

    return k(msg, idx3, zeros_np)


def _tc_lin0(xp, wbd, bt):
    def body(x_ref, w_ref, b_ref, o_ref):
        o_ref[...] = jnp.maximum(
            jnp.dot(x_ref[...], w_ref[...], preferred_element_type=jnp.float32)
            + b_ref[...],
            0.0,
        )

    return pl.pallas_call(
        body,
        out_shape=jax.ShapeDtypeStruct((NPK, 128), jnp.float32),
    )(xp, wbd, bt)


def _tc_wpre(ea, w1, b1, w2, b2):
    def body(ea_ref, w1_ref, b1_ref, w2_ref, b2_ref, o_ref):
        h1 = jnp.maximum(
            jnp.dot(ea_ref[...], w1_ref[...], preferred_element_type=jnp.float32)
            + b1_ref[...],
            0.0,
        )
        o_ref[...] = (
            jnp.dot(h1, w2_ref[...], preferred_element_type=jnp.float32)
            + b2_ref[...]
        )

    return pl.pallas_call(
        body,
        grid=(EP // EB,),
        in_specs=[
            pl.BlockSpec((EB, DE), lambda i: (i, 0)),
            pl.BlockSpec((DE, 64), lambda i: (0, 0)),
            pl.BlockSpec((1, 64), lambda i: (0, 0)),
            pl.BlockSpec((64, H * H), lambda i: (0, 0)),
            pl.BlockSpec((1, H * H), lambda i: (0, 0)),
        ],
        out_specs=pl.BlockSpec((EB, H * H), lambda i: (i, 0)),
        out_shape=jax.ShapeDtypeStruct((EP, H * H), jnp.float32),
    )(ea, w1, b1, w2, b2)


def _tc_msg(gp, w, R, S):
    def body(g_ref, w_ref, R_ref, S_ref, o_ref):
        g = g_ref[...]
        w3 = w_ref[...].reshape(EBK, 8, H * H)
        pieces = []
        for j in range(8):
            gexp = jnp.dot(g[:, 16 * j:16 * j + 16], R_ref[...],
                           preferred_element_type=jnp.float32)
            pieces.append(
                jnp.dot(gexp * w3[:, j, :], S_ref[...],
                        preferred_element_type=jnp.float32)
            )
        o_ref[...] = jnp.concatenate(pieces, axis=1)

    return pl.pallas_call(
        body,
        grid=(EP // EB,),
        in_specs=[
            pl.BlockSpec((EBK, 128), lambda i: (i, 0)),
            pl.BlockSpec((EB, H * H), lambda i: (i, 0)),
            pl.BlockSpec((H, H * H), lambda i: (0, 0)),
            pl.BlockSpec((H * H, H), lambda i: (0, 0)),
        ],
        out_specs=pl.BlockSpec((EBK, 128), lambda i: (i, 0)),
        out_shape=jax.ShapeDtypeStruct((EPK, 128), jnp.float32),
    )(gp, w, R, S)


def _tc_gru(parts, hp, bd_root, cbt, bd_ir, bd_iz, bd_in, bd_hr, bd_hz, bd_hn,
            bih_t, bhh_t):
    def body(p_ref, h_ref, rt_ref, cb_ref, ir_ref, iz_ref, in_ref,
             hr_ref, hz_ref, hn_ref, bi_ref, bh_ref, o_ref):
        h = h_ref[...]
        agg = p_ref[0] + p_ref[1]
        m = jnp.maximum(
            agg
            + jnp.dot(h, rt_ref[...], preferred_element_type=jnp.float32)
            + cb_ref[...],
            0.0,
        )
        bi = bi_ref[...]
        bh = bh_ref[...]
        ir = jnp.dot(m, ir_ref[...], preferred_element_type=jnp.float32) + bi[:, :128]
        iz = jnp.dot(m, iz_ref[...], preferred_element_type=jnp.float32) + bi[:, 128:256]
        inn = jnp.dot(m, in_ref[...], preferred_element_type=jnp.float32) + bi[:, 256:]
        hr = jnp.dot(h, hr_ref[...], preferred_element_type=jnp.float32) + bh[:, :128]
        hz = jnp.dot(h, hz_ref[...], preferred_element_type=jnp.float32) + bh[:, 128:256]
        hn = jnp.dot(h, hn_ref[...], preferred_element_type=jnp.float32) + bh[:, 256:]
        r = jax.nn.sigmoid(ir + hr)
        z = jax.nn.sigmoid(iz + hz)
        n = jnp.tanh(inn + r * hn)
        o_ref[...] = (1.0 - z) * n + z * h

    return pl.pallas_call(
        body,
        out_shape=jax.ShapeDtypeStruct((NPK, 128), jnp.float32),
    )(parts, hp, bd_root, cbt, bd_ir, bd_iz, bd_in, bd_hr, bd_hz, bd_hn,
      bih_t, bhh_t)


def _tc_set2set(out, batch2, t, p, lstm_wih, lstm_whh, lstm_bih, lstm_bhh,
                lin1_w, lin1_b, lin2_w, lin2_b, lin3_w, lin3_b):
    def body(o_ref, b_ref, t_ref, p_ref, wih_ref, whh_ref, bih_ref, bhh_ref,
             w1_ref, b1_ref, w2_ref, b2_ref, w3_ref, b3_ref, res_ref):
        xn = o_ref[...][:N, :]
        mask = b_ref[...] == lax.broadcasted_iota(jnp.int32, (1, B), 1)
        q_star = jnp.zeros((B, 2 * H), jnp.float32)
        hs = jnp.zeros((B, H), jnp.float32)
        cs = jnp.zeros((B, H), jnp.float32)
        for _ in range(3):
            g = (
                jnp.dot(q_star, wih_ref[...], preferred_element_type=jnp.float32)
                + bih_ref[...]
                + jnp.dot(hs, whh_ref[...], preferred_element_type=jnp.float32)
                + bhh_ref[...]
            )
            ig = jax.nn.sigmoid(g[:, :H])
            fg = jax.nn.sigmoid(g[:, H:2 * H])
            gg = jnp.tanh(g[:, 2 * H:3 * H])
            og = jax.nn.sigmoid(g[:, 3 * H:])
            cs = fg * cs + ig * gg
            hs = og * jnp.tanh(cs)
            q = hs
            qb = jnp.dot(
                mask.astype(jnp.float32), q, preferred_element_type=jnp.float32
            )
            e = jnp.sum(xn * qb, axis=1, keepdims=True)
            em = jnp.where(mask, e, -jnp.inf)
            emax = jnp.max(em, axis=0, keepdims=True)
            ee = jnp.where(mask, jnp.exp(e - emax), 0.0)
            den = jnp.sum(ee, axis=0, keepdims=True)
            den = jnp.where(den == 0.0, 1.0, den)
            amat = ee / den
            rr = lax.dot_general(
                amat, xn, (((0,), (0,)), ((), ())),
                preferred_element_type=jnp.float32,
            )
            q_star = jnp.concatenate([q, rr], axis=1)
        o = jnp.concatenate([q_star, t_ref[...], p_ref[...]], axis=1)
        o = jnp.maximum(
            jnp.dot(o, w1_ref[...], preferred_element_type=jnp.float32)
            + b1_ref[...],
            0.0,
        )
        o = jnp.maximum(
            jnp.dot(o, w2_ref[...], preferred_element_type=jnp.float32)
            + b2_ref[...],
            0.0,
        )
        res_ref[...] = (
            jnp.dot(o, w3_ref[...], preferred_element_type=jnp.float32)
            + b3_ref[...]
        )

    return pl.pallas_call(
        body,
        out_shape=jax.ShapeDtypeStruct((B, 1), jnp.float32),
    )(out, batch2, t, p, lstm_wih, lstm_whh, lstm_bih, lstm_bhh,
      lin1_w, lin1_b, lin2_w, lin2_b, lin3_w, lin3_b)


def kernel(x, edge_index, edge_attr, batch, t, p,
           lin0_w, lin0_b, nn1_w, nn1_b, nn2_w, nn2_b, root_w, conv_b,
           gru_wih, gru_whh, gru_bih, gru_bhh,
           lstm_wih, lstm_whh, lstm_bih, lstm_bhh,
           lin1_w, lin1_b, lin2_w, lin2_b, lin3_w, lin3_b):
    f32 = jnp.float32
    src = edge_index[0].astype(jnp.int32)
    dst = edge_index[1].astype(jnp.int32)
    pad = EP - E
    src3 = jnp.concatenate([src, jnp.zeros((pad,), jnp.int32)]).reshape(NW, CH, CL)
    dst3 = jnp.concatenate([dst, jnp.full((pad,), N, jnp.int32)]).reshape(NW, CH, CL)
    x_pad = jnp.concatenate([x, jnp.zeros((NP - N, DIN), f32)])
    zeros_np = jnp.zeros((NP, H), f32)

    R = jnp.kron(jnp.eye(H, dtype=f32), jnp.ones((1, H), f32))
    S = jnp.kron(jnp.ones((H, 1), f32), jnp.eye(H, dtype=f32))
    eye8 = jnp.eye(8, dtype=f32)

    outp = _tc_lin0(
        x_pad.reshape(NPK, 8 * DIN),
        jnp.kron(eye8, lin0_w),
        jnp.tile(lin0_b, 8).reshape(1, 128),
    )

    bd = lambda w: jnp.kron(eye8, w)
    gru_args = (
        bd(root_w), jnp.tile(conv_b, 8).reshape(1, 128),
        bd(gru_wih[:, :H]), bd(gru_wih[:, H:2 * H]), bd(gru_wih[:, 2 * H:]),
        bd(gru_whh[:, :H]), bd(gru_whh[:, H:2 * H]), bd(gru_whh[:, 2 * H:]),
        jnp.concatenate(
            [jnp.tile(gru_bih[g * H:(g + 1) * H], 8) for g in range(3)]
        ).reshape(1, 384),
        jnp.concatenate(
            [jnp.tile(gru_bhh[g * H:(g + 1) * H], 8) for g in range(3)]
        ).reshape(1, 384),
    )

    ea_pad = jnp.concatenate([edge_attr, jnp.zeros((pad, DE), f32)])
    w = _tc_wpre(ea_pad, nn1_w, nn1_b.reshape(1, 64), nn2_w,
                 nn2_b.reshape(1, H * H))
    for _ in range(4):
        g = _sc_gather(outp.reshape(NP, H), src3)
        msgp = _tc_msg(g.reshape(EPK, 128), w, R, S)
        parts = _sc_scatter(msgp.reshape(EP, H), dst3, zeros_np)
        outp = _tc_gru(parts.reshape(NC, NPK, 128), outp, *gru_args)

    res = _tc_set2set(
        outp.reshape(NP, H), batch.astype(jnp.int32).reshape(N, 1), t, p,
        lstm_wih, lstm_whh, lstm_bih.reshape(1, 4 * H), lstm_bhh.reshape(1, 4 * H),
        lin1_w, lin1_b.reshape(1, H), lin2_w, lin2_b.reshape(1, 64),
        lin3_w, lin3_b.reshape(1, 1))
    return res.reshape(-1)


# speedup vs baseline: 5.5830x; 1.1391x over previous
import functools

import jax
import jax.numpy as jnp
from jax import lax
from jax.experimental import pallas as pl
from jax.experimental.pallas import tpu as pltpu
from jax.experimental.pallas import tpu_sc as plsc

N = 10000
E = 160000
DIN = 128
H = 16
B = 64
DE = 6

NC = 2
NS = 16
NW = NC * NS
CL = 128
CH = 40
PW = CH * CL
EP = NW * PW
NP = 10016
RPS = NP // NS

EPK = EP // 8
NPK = NP // 8
EB = 2048
EBK = EB // 8


def _mesh():
    return plsc.VectorSubcoreMesh(core_axis_name="c", subcore_axis_name="s")


def _sc_gather(table, idx3):
    @functools.partial(
        pl.kernel,
        mesh=_mesh(),
        out_type=jax.ShapeDtypeStruct((EP, H), jnp.float32),
        scratch_types=[
            pltpu.VMEM((CH, CL), jnp.int32),
            pltpu.VMEM((PW, H), jnp.float32),
            pltpu.SemaphoreType.DMA,
        ],
        compiler_params=pltpu.CompilerParams(use_tc_tiling_on_sc=False),
    )
    def k(table_hbm, idx_hbm, g_hbm, idx_v, rows_v, sem):
        wid = lax.axis_index("s") * NC + lax.axis_index("c")
        pltpu.sync_copy(idx_hbm.at[wid], idx_v)

        def body(j, carry):
            pltpu.async_copy(
                table_hbm.at[idx_v.at[j]], rows_v.at[pl.ds(j * CL, CL)], sem
            )
            return carry

        lax.fori_loop(0, CH, body, 0)
        pltpu.make_async_copy(g_hbm.at[pl.ds(0, PW)], rows_v, sem).wait()
        pltpu.sync_copy(rows_v, g_hbm.at[pl.ds(wid * PW, PW)])

    return k(table, idx3)


def _sc_scatter(msg, idx3, zeros_np):
    @functools.partial(
        pl.kernel,
        mesh=_mesh(),
        out_type=jax.ShapeDtypeStruct((NC, NP, H), jnp.float32),
        scratch_types=[
            pltpu.VMEM((CH, CL), jnp.int32),
            pltpu.VMEM((PW, H), jnp.float32),
            pltpu.VMEM_SHARED((NP, H), jnp.float32),
            pltpu.SemaphoreType.DMA,
        ],
        compiler_params=pltpu.CompilerParams(use_tc_tiling_on_sc=False),
    )
    def k(msg_hbm, idx_hbm, z_hbm, parts_hbm, idx_v, msg_v, agg_sh, sem):
        cid = lax.axis_index("c")
        sid = lax.axis_index("s")
        wid = sid * NC + cid
        pltpu.sync_copy(
            z_hbm.at[pl.ds(sid * RPS, RPS)], agg_sh.at[pl.ds(sid * RPS, RPS)]
        )
        pltpu.sync_copy(idx_hbm.at[wid], idx_v)
        pltpu.sync_copy(msg_hbm.at[pl.ds(wid * PW, PW)], msg_v)
        plsc.subcore_barrier()

        def body(j, carry):
            pltpu.async_copy(
                msg_v.at[pl.ds(j * CL, CL)], agg_sh.at[idx_v.at[j]], sem,
                add=True,
            )
            return carry

        lax.fori_loop(0, CH, body, 0)
        pltpu.make_async_copy(msg_hbm.at[pl.ds(0, PW)], msg_v, sem).wait()
        plsc.subcore_barrier()
        pltpu.sync_copy(
            agg_sh.at[pl.ds(sid * RPS, RPS)],
            parts_hbm.at[cid, pl.ds(sid * RPS, RPS)],
        )

    return k(msg, idx3, zeros_np)


def _tc_lin0(xp, wbd, bt):
    def body(x_ref, w_ref, b_ref, o_ref):
        o_ref[...] = jnp.maximum(
            jnp.dot(x_ref[...], w_ref[...], preferred_element_type=jnp.float32)
            + b_ref[...],
            0.0,
        )

    return pl.pallas_call(
        body,
        out_shape=jax.ShapeDtypeStruct((NPK, 128), jnp.float32),
    )(xp, wbd, bt)


def _tc_wpre(eaT, w1, b1, w2, b2):
    def body(ea_ref, w1_ref, b1_ref, w2_ref, b2_ref, o_ref):
        h1 = jnp.maximum(
            lax.dot_general(ea_ref[...], w1_ref[...], (((0,), (0,)), ((), ())),
                            preferred_element_type=jnp.float32)
            + b1_ref[...],
            0.0,
        )
        o_ref[...] = (
            jnp.dot(h1, w2_ref[...], preferred_element_type=jnp.float32)
            + b2_ref[...]
        ).astype(jnp.bfloat16)

    return pl.pallas_call(
        body,
        grid=(EP // EB,),
        in_specs=[
            pl.BlockSpec((DE, EB), lambda i: (0, i)),
            pl.BlockSpec((DE, 64), lambda i: (0, 0)),
            pl.BlockSpec((1, 64), lambda i: (0, 0)),
            pl.BlockSpec((64, H * H), lambda i: (0, 0)),
            pl.BlockSpec((1, H * H), lambda i: (0, 0)),
        ],
        out_specs=pl.BlockSpec((EB, H * H), lambda i: (i, 0)),
        out_shape=jax.ShapeDtypeStruct((EP, H * H), jnp.bfloat16),
    )(eaT, w1, b1, w2, b2)


def _tc_msg(gp, w, R, S):
    def body(g_ref, w_ref, R_ref, S_ref, o_ref):
        g16 = g_ref[...].astype(jnp.bfloat16)
        w3 = w_ref[...].reshape(EBK, 8, H * H)
        pieces = []
        for j in range(8):
            gexp = jnp.dot(g16[:, 16 * j:16 * j + 16], R_ref[...],
                           preferred_element_type=jnp.float32).astype(jnp.bfloat16)
            pieces.append(
                jnp.dot(gexp * w3[:, j, :], S_ref[...],
                        preferred_element_type=jnp.float32)
            )
        o_ref[...] = jnp.concatenate(pieces, axis=1)

    return pl.pallas_call(
        body,
        grid=(EP // EB,),
        in_specs=[
            pl.BlockSpec((EBK, 128), lambda i: (i, 0)),
            pl.BlockSpec((EB, H * H), lambda i: (i, 0)),
            pl.BlockSpec((H, H * H), lambda i: (0, 0)),
            pl.BlockSpec((H * H, H), lambda i: (0, 0)),
        ],
        out_specs=pl.BlockSpec((EBK, 128), lambda i: (i, 0)),
        out_shape=jax.ShapeDtypeStruct((EPK, 128), jnp.float32),
    )(gp, w, R, S)


def _tc_gru(parts, hp, bd_root, cbt, bd_ir, bd_iz, bd_in, bd_hr, bd_hz, bd_hn,
            bih_t, bhh_t):
    def body(p_ref, h_ref, rt_ref, cb_ref, ir_ref, iz_ref, in_ref,
             hr_ref, hz_ref, hn_ref, bi_ref, bh_ref, o_ref):
        h = h_ref[...]
        agg = p_ref[0] + p_ref[1]
        m = jnp.maximum(
            agg
            + jnp.dot(h, rt_ref[...], preferred_element_type=jnp.float32)
            + cb_ref[...],
            0.0,
        )
        bi = bi_ref[...]
        bh = bh_ref[...]
        ir = jnp.dot(m, ir_ref[...], preferred_element_type=jnp.float32) + bi[:, :128]
        iz = jnp.dot(m, iz_ref[...], preferred_element_type=jnp.float32) + bi[:, 128:256]
        inn = jnp.dot(m, in_ref[...], preferred_element_type=jnp.float32) + bi[:, 256:]
        hr = jnp.dot(h, hr_ref[...], preferred_element_type=jnp.float32) + bh[:, :128]
        hz = jnp.dot(h, hz_ref[...], preferred_element_type=jnp.float32) + bh[:, 128:256]
        hn = jnp.dot(h, hn_ref[...], preferred_element_type=jnp.float32) + bh[:, 256:]
        r = jax.nn.sigmoid(ir + hr)
        z = jax.nn.sigmoid(iz + hz)
        n = jnp.tanh(inn + r * hn)
        o_ref[...] = (1.0 - z) * n + z * h

    return pl.pallas_call(
        body,
        out_shape=jax.ShapeDtypeStruct((NPK, 128), jnp.float32),
    )(parts, hp, bd_root, cbt, bd_ir, bd_iz, bd_in, bd_hr, bd_hz, bd_hn,
      bih_t, bhh_t)


def _tc_set2set(out, batch2, t, p, lstm_wih, lstm_whh, lstm_bih, lstm_bhh,
                lin1_w, lin1_b, lin2_w, lin2_b, lin3_w, lin3_b):
    def body(o_ref, b_ref, t_ref, p_ref, wih_ref, whh_ref, bih_ref, bhh_ref,
             w1_ref, b1_ref, w2_ref, b2_ref, w3_ref, b3_ref, res_ref):
        xn = o_ref[...][:N, :]
        mask = b_ref[...] == lax.broadcasted_iota(jnp.int32, (1, B), 1)
        q_star = jnp.zeros((B, 2 * H), jnp.float32)
        hs = jnp.zeros((B, H), jnp.float32)
        cs = jnp.zeros((B, H), jnp.float32)
        for _ in range(3):
            g = (
                jnp.dot(q_star, wih_ref[...], preferred_element_type=jnp.float32)
                + bih_ref[...]
                + jnp.dot(hs, whh_ref[...], preferred_element_type=jnp.float32)
                + bhh_ref[...]
            )
            ig = jax.nn.sigmoid(g[:, :H])
            fg = jax.nn.sigmoid(g[:, H:2 * H])
            gg = jnp.tanh(g[:, 2 * H:3 * H])
            og = jax.nn.sigmoid(g[:, 3 * H:])
            cs = fg * cs + ig * gg
            hs = og * jnp.tanh(cs)
            q = hs
            qb = jnp.dot(
                mask.astype(jnp.float32), q, preferred_element_type=jnp.float32
            )
            e = jnp.sum(xn * qb, axis=1, keepdims=True)
            em = jnp.where(mask, e, -jnp.inf)
            emax = jnp.max(em, axis=0, keepdims=True)
            ee = jnp.where(mask, jnp.exp(e - emax), 0.0)
            den = jnp.sum(ee, axis=0, keepdims=True)
            den = jnp.where(den == 0.0, 1.0, den)
            amat = ee / den
            rr = lax.dot_general(
                amat, xn, (((0,), (0,)), ((), ())),
                preferred_element_type=jnp.float32,
            )
            q_star = jnp.concatenate([q, rr], axis=1)
        o = jnp.concatenate([q_star, t_ref[...], p_ref[...]], axis=1)
        o = jnp.maximum(
            jnp.dot(o, w1_ref[...], preferred_element_type=jnp.float32)
            + b1_ref[...],
            0.0,
        )
        o = jnp.maximum(
            jnp.dot(o, w2_ref[...], preferred_element_type=jnp.float32)
            + b2_ref[...],
            0.0,
        )
        res_ref[...] = (
            jnp.dot(o, w3_ref[...], preferred_element_type=jnp.float32)
            + b3_ref[...]
        )

    return pl.pallas_call(
        body,
        out_shape=jax.ShapeDtypeStruct((B, 1), jnp.float32),
    )(out, batch2, t, p, lstm_wih, lstm_whh, lstm_bih, lstm_bhh,
      lin1_w, lin1_b, lin2_w, lin2_b, lin3_w, lin3_b)


def kernel(x, edge_index, edge_attr, batch, t, p,
           lin0_w, lin0_b, nn1_w, nn1_b, nn2_w, nn2_b, root_w, conv_b,
           gru_wih, gru_whh, gru_bih, gru_bhh,
           lstm_wih, lstm_whh, lstm_bih, lstm_bhh,
           lin1_w, lin1_b, lin2_w, lin2_b, lin3_w, lin3_b):
    f32 = jnp.float32
    src = edge_index[0].astype(jnp.int32)
    dst = edge_index[1].astype(jnp.int32)
    pad = EP - E
    src3 = jnp.concatenate([src, jnp.zeros((pad,), jnp.int32)]).reshape(NW, CH, CL)
    dst3 = jnp.concatenate([dst, jnp.full((pad,), N, jnp.int32)]).reshape(NW, CH, CL)
    x_pad = jnp.concatenate([x, jnp.zeros((NP - N, DIN), f32)])
    zeros_np = jnp.zeros((NP, H), f32)

    R = jnp.kron(jnp.eye(H, dtype=f32), jnp.ones((1, H), f32))
    S = jnp.kron(jnp.ones((H, 1), f32), jnp.eye(H, dtype=f32))
    R16 = R.astype(jnp.bfloat16)
    S16 = S.astype(jnp.bfloat16)
    eye8 = jnp.eye(8, dtype=f32)

    outp = _tc_lin0(
        x_pad.reshape(NPK, 8 * DIN),
        jnp.kron(eye8, lin0_w),
        jnp.tile(lin0_b, 8).reshape(1, 128),
    )

    bd = lambda w: jnp.kron(eye8, w)
    gru_args = (
        bd(root_w), jnp.tile(conv_b, 8).reshape(1, 128),
        bd(gru_wih[:, :H]), bd(gru_wih[:, H:2 * H]), bd(gru_wih[:, 2 * H:]),
        bd(gru_whh[:, :H]), bd(gru_whh[:, H:2 * H]), bd(gru_whh[:, 2 * H:]),
        jnp.concatenate(
            [jnp.tile(gru_bih[g * H:(g + 1) * H], 8) for g in range(3)]
        ).reshape(1, 384),
        jnp.concatenate(
            [jnp.tile(gru_bhh[g * H:(g + 1) * H], 8) for g in range(3)]
        ).reshape(1, 384),
    )

    eaT = jnp.concatenate([edge_attr.T, jnp.zeros((DE, pad), f32)], axis=1)
    w = _tc_wpre(eaT, nn1_w, nn1_b.reshape(1, 64), nn2_w,
                 nn2_b.reshape(1, H * H))
    for _ in range(4):
        g = _sc_gather(outp.reshape(NP, H), src3)
        msgp = _tc_msg(g.reshape(EPK, 128), w, R16, S16)
        parts = _sc_scatter(msgp.reshape(EP, H), dst3, zeros_np)
        outp = _tc_gru(parts.reshape(NC, NPK, 128), outp, *gru_args)

    res = _tc_set2set(
        outp.reshape(NP, H), batch.astype(jnp.int32).reshape(N, 1), t, p,
        lstm_wih, lstm_whh, lstm_bih.reshape(1, 4 * H), lstm_bhh.reshape(1, 4 * H),
        lin1_w, lin1_b.reshape(1, H), lin2_w, lin2_b.reshape(1, 64),
        lin3_w, lin3_b.reshape(1, 1))
    return res.reshape(-1)


# speedup vs baseline: 5.6852x; 1.0183x over previous
import functools

import jax
import jax.numpy as jnp
from jax import lax
from jax.experimental import pallas as pl
from jax.experimental.pallas import tpu as pltpu
from jax.experimental.pallas import tpu_sc as plsc

N = 10000
E = 160000
DIN = 128
H = 16
B = 64
DE = 6

NC = 2
NS = 16
NW = NC * NS
CL = 128
CH = 40
PW = CH * CL
EP = NW * PW
NP = 10016
RPS = NP // NS

EPK = EP // 8
NPK = NP // 8
EB = 2048
EBK = EB // 8


def _mesh():
    return plsc.VectorSubcoreMesh(core_axis_name="c", subcore_axis_name="s")


def _sc_gather(table, idx3):
    @functools.partial(
        pl.kernel,
        mesh=_mesh(),
        out_type=jax.ShapeDtypeStruct((EP, H), jnp.float32),
        scratch_types=[
            pltpu.VMEM((CH, CL), jnp.int32),
            pltpu.VMEM((PW, H), jnp.float32),
            pltpu.SemaphoreType.DMA,
        ],
        compiler_params=pltpu.CompilerParams(use_tc_tiling_on_sc=False),
    )
    def k(table_hbm, idx_hbm, g_hbm, idx_v, rows_v, sem):
        wid = lax.axis_index("s") * NC + lax.axis_index("c")
        pltpu.sync_copy(idx_hbm.at[wid], idx_v)

        def body(j, carry):
            pltpu.async_copy(
                table_hbm.at[idx_v.at[j]], rows_v.at[pl.ds(j * CL, CL)], sem
            )
            return carry

        lax.fori_loop(0, CH, body, 0)
        pltpu.make_async_copy(g_hbm.at[pl.ds(0, PW)], rows_v, sem).wait()
        pltpu.sync_copy(rows_v, g_hbm.at[pl.ds(wid * PW, PW)])

    return k(table, idx3)


def _sc_scatter(msg, idx3, zeros_np):
    @functools.partial(
        pl.kernel,
        mesh=_mesh(),
        out_type=jax.ShapeDtypeStruct((NC, NP, H), jnp.float32),
        scratch_types=[
            pltpu.VMEM((CH, CL), jnp.int32),
            pltpu.VMEM((PW, H), jnp.float32),
            pltpu.VMEM_SHARED((NP, H), jnp.float32),
            pltpu.SemaphoreType.DMA,
        ],
        compiler_params=pltpu.CompilerParams(use_tc_tiling_on_sc=False),
    )
    def k(msg_hbm, idx_hbm, z_hbm, parts_hbm, idx_v, msg_v, agg_sh, sem):
        cid = lax.axis_index("c")
        sid = lax.axis_index("s")
        wid = sid * NC + cid
        pltpu.sync_copy(
            z_hbm.at[pl.ds(sid * RPS, RPS)], agg_sh.at[pl.ds(sid * RPS, RPS)]
        )
        pltpu.sync_copy(idx_hbm.at[wid], idx_v)
        pltpu.sync_copy(msg_hbm.at[pl.ds(wid * PW, PW)], msg_v)
        plsc.subcore_barrier()

        def body(j, carry):
            pltpu.async_copy(
                msg_v.at[pl.ds(j * CL, CL)], agg_sh.at[idx_v.at[j]], sem,
                add=True,
            )
            return carry

        lax.fori_loop(0, CH, body, 0)
        pltpu.make_async_copy(msg_hbm.at[pl.ds(0, PW)], msg_v, sem).wait()
        plsc.subcore_barrier()
        pltpu.sync_copy(
            agg_sh.at[pl.ds(sid * RPS, RPS)],
            parts_hbm.at[cid, pl.ds(sid * RPS, RPS)],
        )

    return k(msg, idx3, zeros_np)


def _tc_lin0(xp, wbd, bt):
    def body(x_ref, w_ref, b_ref, o_ref):
        o_ref[...] = jnp.maximum(
            jnp.dot(x_ref[...], w_ref[...], preferred_element_type=jnp.float32)
            + b_ref[...],
            0.0,
        )

    return pl.pallas_call(
        body,
        out_shape=jax.ShapeDtypeStruct((NPK, 128), jnp.float32),
    )(xp, wbd, bt)


def _tc_wpre(eaT, w1, b1, w2, b2):
    def body(ea_ref, w1_ref, b1_ref, w2_ref, b2_ref, o_ref):
        h1 = jnp.maximum(
            lax.dot_general(ea_ref[...], w1_ref[...], (((0,), (0,)), ((), ())),
                            preferred_element_type=jnp.float32)
            + b1_ref[...],
            0.0,
        )
        o_ref[...] = (
            jnp.dot(h1, w2_ref[...], preferred_element_type=jnp.float32)
            + b2_ref[...]
        ).astype(jnp.bfloat16)

    return pl.pallas_call(
        body,
        grid=(EP // EB,),
        in_specs=[
            pl.BlockSpec((DE, EB), lambda i: (0, i)),
            pl.BlockSpec((DE, 64), lambda i: (0, 0)),
            pl.BlockSpec((1, 64), lambda i: (0, 0)),
            pl.BlockSpec((64, H * H), lambda i: (0, 0)),
            pl.BlockSpec((1, H * H), lambda i: (0, 0)),
        ],
        out_specs=pl.BlockSpec((EB, H * H), lambda i: (i, 0)),
        out_shape=jax.ShapeDtypeStruct((EP, H * H), jnp.bfloat16),
    )(eaT, w1, b1, w2, b2)


def _tc_msg(gp, w, R, S):
    def body(g_ref, w_ref, R_ref, S_ref, o_ref):
        g16 = g_ref[...].astype(jnp.bfloat16)
        w = w_ref[...]
        pieces = []
        for j in range(8):
            gexp = jnp.dot(g16[:, 16 * j:16 * j + 16], R_ref[...],
                           preferred_element_type=jnp.float32).astype(jnp.bfloat16)
            pieces.append(
                jnp.dot(gexp * w[EBK * j:EBK * (j + 1), :], S_ref[...],
                        preferred_element_type=jnp.float32)
            )
        o_ref[...] = jnp.concatenate(pieces, axis=1)

    return pl.pallas_call(
        body,
        grid=(EP // EB,),
        in_specs=[
            pl.BlockSpec((EBK, 128), lambda i: (i, 0)),
            pl.BlockSpec((EB, H * H), lambda i: (i, 0)),
            pl.BlockSpec((H, H * H), lambda i: (0, 0)),
            pl.BlockSpec((H * H, H), lambda i: (0, 0)),
        ],
        out_specs=pl.BlockSpec((EBK, 128), lambda i: (i, 0)),
        out_shape=jax.ShapeDtypeStruct((EPK, 128), jnp.float32),
    )(gp, w, R, S)


def _tc_gru(parts, hp, bd_root, cbt, bd_ir, bd_iz, bd_in, bd_hr, bd_hz, bd_hn,
            bih_t, bhh_t):
    def body(p_ref, h_ref, rt_ref, cb_ref, ir_ref, iz_ref, in_ref,
             hr_ref, hz_ref, hn_ref, bi_ref, bh_ref, o_ref):
        h = h_ref[...]
        agg = p_ref[0] + p_ref[1]
        m = jnp.maximum(
            agg
            + jnp.dot(h, rt_ref[...], preferred_element_type=jnp.float32)
            + cb_ref[...],
            0.0,
        )
        bi = bi_ref[...]
        bh = bh_ref[...]
        ir = jnp.dot(m, ir_ref[...], preferred_element_type=jnp.float32) + bi[:, :128]
        iz = jnp.dot(m, iz_ref[...], preferred_element_type=jnp.float32) + bi[:, 128:256]
        inn = jnp.dot(m, in_ref[...], preferred_element_type=jnp.float32) + bi[:, 256:]
        hr = jnp.dot(h, hr_ref[...], preferred_element_type=jnp.float32) + bh[:, :128]
        hz = jnp.dot(h, hz_ref[...], preferred_element_type=jnp.float32) + bh[:, 128:256]
        hn = jnp.dot(h, hn_ref[...], preferred_element_type=jnp.float32) + bh[:, 256:]
        r = jax.nn.sigmoid(ir + hr)
        z = jax.nn.sigmoid(iz + hz)
        n = jnp.tanh(inn + r * hn)
        o_ref[...] = (1.0 - z) * n + z * h

    return pl.pallas_call(
        body,
        out_shape=jax.ShapeDtypeStruct((NPK, 128), jnp.float32),
    )(parts, hp, bd_root, cbt, bd_ir, bd_iz, bd_in, bd_hr, bd_hz, bd_hn,
      bih_t, bhh_t)


def _tc_set2set(out, batch2, t, p, lstm_wih, lstm_whh, lstm_bih, lstm_bhh,
                lin1_w, lin1_b, lin2_w, lin2_b, lin3_w, lin3_b):
    def body(o_ref, b_ref, t_ref, p_ref, wih_ref, whh_ref, bih_ref, bhh_ref,
             w1_ref, b1_ref, w2_ref, b2_ref, w3_ref, b3_ref, res_ref):
        xn = o_ref[...][:N, :]
        mask = b_ref[...] == lax.broadcasted_iota(jnp.int32, (1, B), 1)
        q_star = jnp.zeros((B, 2 * H), jnp.float32)
        hs = jnp.zeros((B, H), jnp.float32)
        cs = jnp.zeros((B, H), jnp.float32)
        for _ in range(3):
            g = (
                jnp.dot(q_star, wih_ref[...], preferred_element_type=jnp.float32)
                + bih_ref[...]
                + jnp.dot(hs, whh_ref[...], preferred_element_type=jnp.float32)
                + bhh_ref[...]
            )
            ig = jax.nn.sigmoid(g[:, :H])
            fg = jax.nn.sigmoid(g[:, H:2 * H])
            gg = jnp.tanh(g[:, 2 * H:3 * H])
            og = jax.nn.sigmoid(g[:, 3 * H:])
            cs = fg * cs + ig * gg
            hs = og * jnp.tanh(cs)
            q = hs
            qb = jnp.dot(
                mask.astype(jnp.float32), q, preferred_element_type=jnp.float32
            )
            e = jnp.sum(xn * qb, axis=1, keepdims=True)
            em = jnp.where(mask, e, -jnp.inf)
            emax = jnp.max(em, axis=0, keepdims=True)
            ee = jnp.where(mask, jnp.exp(e - emax), 0.0)
            den = jnp.sum(ee, axis=0, keepdims=True)
            den = jnp.where(den == 0.0, 1.0, den)
            amat = ee / den
            rr = lax.dot_general(
                amat, xn, (((0,), (0,)), ((), ())),
                preferred_element_type=jnp.float32,
            )
            q_star = jnp.concatenate([q, rr], axis=1)
        o = jnp.concatenate([q_star, t_ref[...], p_ref[...]], axis=1)
        o = jnp.maximum(
            jnp.dot(o, w1_ref[...], preferred_element_type=jnp.float32)
            + b1_ref[...],
            0.0,
        )
        o = jnp.maximum(
            jnp.dot(o, w2_ref[...], preferred_element_type=jnp.float32)
            + b2_ref[...],
            0.0,
        )
        res_ref[...] = (
            jnp.dot(o, w3_ref[...], preferred_element_type=jnp.float32)
            + b3_ref[...]
        )

    return pl.pallas_call(
        body,
        out_shape=jax.ShapeDtypeStruct((B, 1), jnp.float32),
    )(out, batch2, t, p, lstm_wih, lstm_whh, lstm_bih, lstm_bhh,
      lin1_w, lin1_b, lin2_w, lin2_b, lin3_w, lin3_b)


def kernel(x, edge_index, edge_attr, batch, t, p,
           lin0_w, lin0_b, nn1_w, nn1_b, nn2_w, nn2_b, root_w, conv_b,
           gru_wih, gru_whh, gru_bih, gru_bhh,
           lstm_wih, lstm_whh, lstm_bih, lstm_bhh,
           lin1_w, lin1_b, lin2_w, lin2_b, lin3_w, lin3_b):
    f32 = jnp.float32
    src = edge_index[0].astype(jnp.int32)
    dst = edge_index[1].astype(jnp.int32)
    pad = EP - E

    src3 = jnp.concatenate([src, jnp.zeros((pad,), jnp.int32)]).reshape(NW, CH, CL)
    dst3 = jnp.concatenate([dst, jnp.full((pad,), N, jnp.int32)]).reshape(NW, CH, CL)
    x_pad = jnp.concatenate([x, jnp.zeros((NP - N, DIN), f32)])
    zeros_np = jnp.zeros((NP, H), f32)

    R = jnp.kron(jnp.eye(H, dtype=f32), jnp.ones((1, H), f32))
    S = jnp.kron(jnp.ones((H, 1), f32), jnp.eye(H, dtype=f32))
    R16 = R.astype(jnp.bfloat16)
    S16 = S.astype(jnp.bfloat16)
    eye8 = jnp.eye(8, dtype=f32)

    outp = _tc_lin0(
        x_pad.reshape(NPK, 8 * DIN),
        jnp.kron(eye8, lin0_w),
        jnp.tile(lin0_b, 8).reshape(1, 128),
    )

    bd = lambda w: jnp.kron(eye8, w)
    gru_args = (
        bd(root_w), jnp.tile(conv_b, 8).reshape(1, 128),
        bd(gru_wih[:, :H]), bd(gru_wih[:, H:2 * H]), bd(gru_wih[:, 2 * H:]),
        bd(gru_whh[:, :H]), bd(gru_whh[:, H:2 * H]), bd(gru_whh[:, 2 * H:]),
        jnp.concatenate(
            [jnp.tile(gru_bih[g * H:(g + 1) * H], 8) for g in range(3)]
        ).reshape(1, 384),
        jnp.concatenate(
            [jnp.tile(gru_bhh[g * H:(g + 1) * H], 8) for g in range(3)]
        ).reshape(1, 384),
    )

    eaT = jnp.concatenate([edge_attr.T, jnp.zeros((DE, pad), f32)], axis=1)
    eaT = eaT.reshape(DE, EP // EB, EBK, 8).swapaxes(2, 3).reshape(DE, EP)
    w = _tc_wpre(eaT, nn1_w, nn1_b.reshape(1, 64), nn2_w,
                 nn2_b.reshape(1, H * H))
    for _ in range(4):
        g = _sc_gather(outp.reshape(NP, H), src3)
        msgp = _tc_msg(g.reshape(EPK, 128), w, R16, S16)
        parts = _sc_scatter(msgp.reshape(EP, H), dst3, zeros_np)
        outp = _tc_gru(parts.reshape(NC, NPK, 128), outp, *gru_args)

    res = _tc_set2set(
        outp.reshape(NP, H), batch.astype(jnp.int32).reshape(N, 1), t, p,
        lstm_wih, lstm_whh, lstm_bih.reshape(1, 4 * H), lstm_bhh.reshape(1, 4 * H),
        lin1_w, lin1_b.reshape(1, H), lin2_w, lin2_b.reshape(1, 64),
        lin3_w, lin3_b.reshape(1, 1))
    return res.reshape(-1)
